# baseline (device time: 453502 ns/iter reference)
import jax
import jax.numpy as jnp
from jax import lax
from jax.experimental import pallas as pl
from jax.experimental.pallas import tpu as pltpu

N_DEV = 32
F32 = jnp.float32


def kernel(x, Wq, K_ext, V_ext, Wo):
    B, Sq, Dm = x.shape
    _, Skv, Hq, Dh = K_ext.shape
    HD = Hq * Dh
    K2 = K_ext.reshape(B, Skv, HD)
    V2 = V_ext.reshape(B, Skv, HD)

    def body(x_ref, wq_ref, k_ref, v_ref, wo_ref, out_ref,
             q_sc, acc_sc, l_sc, kcomm, vcomm,
             ksend, krecv, vsend, vrecv):
        my = lax.axis_index("i")
        left = lax.rem(my - 1 + N_DEV, N_DEV)
        right = lax.rem(my + 1, N_DEV)

        barrier = pltpu.get_barrier_semaphore()
        pl.semaphore_signal(barrier, inc=1, device_id=(left,),
                            device_id_type=pl.DeviceIdType.MESH)
        pl.semaphore_signal(barrier, inc=1, device_id=(right,),
                            device_id_type=pl.DeviceIdType.MESH)
        pl.semaphore_wait(barrier, 2)

        for b in range(B):
            q_sc[b] = (jnp.dot(x_ref[b], wq_ref[...],
                               preferred_element_type=F32) * 0.125)
            acc_sc[b] = jnp.zeros((Sq, HD), F32)
        l_sc[...] = jnp.zeros((B, Hq, Sq, 1), F32)

        qb = lax.broadcasted_iota(jnp.int32, (Sq, Skv), 0) // 64
        kb = lax.broadcasted_iota(jnp.int32, (Sq, Skv), 1) // 64
        mask = qb == kb

        def attend(kblk, vblk):
            for b in range(B):
                for h in range(Hq):
                    qh = q_sc[b, :, h * Dh:(h + 1) * Dh]
                    kh = kblk[b][:, h * Dh:(h + 1) * Dh]
                    s = lax.dot_general(qh, kh, (((1,), (1,)), ((), ())),
                                        preferred_element_type=F32)
                    w = jnp.where(mask, jnp.exp(s), 0.0)
                    l_sc[b, h] += jnp.sum(w, axis=1, keepdims=True)
                    acc_sc[b, :, h * Dh:(h + 1) * Dh] += jnp.dot(
                        w, vblk[b][:, h * Dh:(h + 1) * Dh],
                        preferred_element_type=F32)

        attend([k_ref[b] for b in range(B)], [v_ref[b] for b in range(B)])
        kcomm[0] = k_ref[...]
        vcomm[0] = v_ref[...]

        def hop(send_slot, recv_slot):
            krdma = pltpu.make_async_remote_copy(
                src_ref=kcomm.at[send_slot], dst_ref=kcomm.at[recv_slot],
                send_sem=ksend.at[send_slot], recv_sem=krecv.at[recv_slot],
                device_id=(right,), device_id_type=pl.DeviceIdType.MESH)
            vrdma = pltpu.make_async_remote_copy(
                src_ref=vcomm.at[send_slot], dst_ref=vcomm.at[recv_slot],
                send_sem=vsend.at[send_slot], recv_sem=vrecv.at[recv_slot],
                device_id=(right,), device_id_type=pl.DeviceIdType.MESH)
            krdma.start()
            vrdma.start()
            krdma.wait()
            vrdma.wait()
            attend([kcomm[recv_slot, b] for b in range(B)],
                   [vcomm[recv_slot, b] for b in range(B)])

        def pair(t, carry):
            hop(0, 1)
            hop(1, 0)
            return carry
        lax.fori_loop(0, (N_DEV - 1) // 2, pair, 0)
        hop(0, 1)

        for b in range(B):
            for h in range(Hq):
                acc_sc[b, :, h * Dh:(h + 1) * Dh] = (
                    acc_sc[b, :, h * Dh:(h + 1) * Dh] / l_sc[b, h])
            out_ref[b] = jnp.dot(acc_sc[b], wo_ref[...],
                                 preferred_element_type=F32)

    return pl.pallas_call(
        body,
        out_shape=jax.ShapeDtypeStruct((B, Sq, Dm), F32),
        in_specs=[pl.BlockSpec(memory_space=pltpu.VMEM)] * 5,
        out_specs=pl.BlockSpec(memory_space=pltpu.VMEM),
        scratch_shapes=[
            pltpu.VMEM((B, Sq, HD), F32),
            pltpu.VMEM((B, Sq, HD), F32),
            pltpu.VMEM((B, Hq, Sq, 1), F32),
            pltpu.VMEM((2, B, Skv, HD), F32),
            pltpu.VMEM((2, B, Skv, HD), F32),
            pltpu.SemaphoreType.DMA((2,)),
            pltpu.SemaphoreType.DMA((2,)),
            pltpu.SemaphoreType.DMA((2,)),
            pltpu.SemaphoreType.DMA((2,)),
        ],
        compiler_params=pltpu.CompilerParams(collective_id=0),
    )(x, Wq, K2, V2, Wo)


# device time: 274501 ns/iter; 1.6521x vs baseline; 1.6521x over previous
import jax
import jax.numpy as jnp
from jax import lax
from jax.experimental import pallas as pl
from jax.experimental.pallas import tpu as pltpu

N_DEV = 32
F32 = jnp.float32
BF16 = jnp.bfloat16


def kernel(x, Wq, K_ext, V_ext, Wo):
    B, Sq, Dm = x.shape
    _, Skv, Hq, Dh = K_ext.shape
    HD = Hq * Dh
    K2 = K_ext.reshape(B, Skv, HD).astype(BF16)
    V2 = V_ext.reshape(B, Skv, HD).astype(BF16)

    def body(x_ref, wq_ref, k_ref, v_ref, wo_ref, out_ref,
             q_sc, acc_sc, l_sc,
             kR, vR, kL, vL,
             kRs, kRr, vRs, vRr, kLs, kLr, vLs, vLr):
        my = lax.axis_index("i")
        left = lax.rem(my - 1 + N_DEV, N_DEV)
        right = lax.rem(my + 1, N_DEV)

        barrier = pltpu.get_barrier_semaphore()
        pl.semaphore_signal(barrier, inc=1, device_id=(left,),
                            device_id_type=pl.DeviceIdType.MESH)
        pl.semaphore_signal(barrier, inc=1, device_id=(right,),
                            device_id_type=pl.DeviceIdType.MESH)
        pl.semaphore_wait(barrier, 2)

        for b in range(B):
            q_sc[b] = (jnp.dot(x_ref[b], wq_ref[...],
                               preferred_element_type=F32) * 0.125)
            acc_sc[b] = jnp.zeros((Sq, HD), F32)
        l_sc[...] = jnp.zeros((B, Hq, Sq, 1), F32)

        qb = lax.broadcasted_iota(jnp.int32, (Sq, Skv), 0) // 64
        kb = lax.broadcasted_iota(jnp.int32, (Sq, Skv), 1) // 64
        mask = qb == kb

        def attend(kvs):
            for b, (k32, v32) in enumerate(kvs):
                for h in range(Hq):
                    qh = q_sc[b, :, h * Dh:(h + 1) * Dh]
                    kh = k32[:, h * Dh:(h + 1) * Dh]
                    s = lax.dot_general(qh, kh, (((1,), (1,)), ((), ())),
                                        preferred_element_type=F32)
                    w = jnp.where(mask, jnp.exp(s), 0.0)
                    l_sc[b, h] += jnp.sum(w, axis=1, keepdims=True)
                    acc_sc[b, :, h * Dh:(h + 1) * Dh] += jnp.dot(
                        w, v32[:, h * Dh:(h + 1) * Dh],
                        preferred_element_type=F32)

        attend([(k_ref[b].astype(F32), v_ref[b].astype(F32))
                for b in range(B)])
        kR[0] = k_ref[...]
        vR[0] = v_ref[...]
        kL[0] = k_ref[...]
        vL[0] = v_ref[...]

        def mk(buf, ssem, rsem, ss, rs, dev):
            return pltpu.make_async_remote_copy(
                src_ref=buf.at[ss], dst_ref=buf.at[rs],
                send_sem=ssem.at[ss], recv_sem=rsem.at[rs],
                device_id=(dev,), device_id_type=pl.DeviceIdType.MESH)

        def start_r(ss, rs):
            a = mk(kR, kRs, kRr, ss, rs, right)
            b_ = mk(vR, vRs, vRr, ss, rs, right)
            a.start()
            b_.start()
            return a, b_

        def start_l(ss, rs):
            a = mk(kL, kLs, kLr, ss, rs, left)
            b_ = mk(vL, vLs, vLr, ss, rs, left)
            a.start()
            b_.start()
            return a, b_

        def both(ss, rs):
            rd = start_r(ss, rs) + start_l(ss, rs)
            for d in rd:
                d.wait()
            attend([(kR[rs, b].astype(F32), vR[rs, b].astype(F32))
                    for b in range(B)])
            attend([(kL[rs, b].astype(F32), vL[rs, b].astype(F32))
                    for b in range(B)])

        def pair(t, carry):
            both(0, 1)
            both(1, 0)
            return carry
        lax.fori_loop(0, 7, pair, 0)
        both(0, 1)
        rd = start_r(1, 0)
        for d in rd:
            d.wait()
        attend([(kR[0, b].astype(F32), vR[0, b].astype(F32))
                for b in range(B)])

        for b in range(B):
            for h in range(Hq):
                acc_sc[b, :, h * Dh:(h + 1) * Dh] = (
                    acc_sc[b, :, h * Dh:(h + 1) * Dh] / l_sc[b, h])
            out_ref[b] = jnp.dot(acc_sc[b], wo_ref[...],
                                 preferred_element_type=F32)

    return pl.pallas_call(
        body,
        out_shape=jax.ShapeDtypeStruct((B, Sq, Dm), F32),
        in_specs=[pl.BlockSpec(memory_space=pltpu.VMEM)] * 5,
        out_specs=pl.BlockSpec(memory_space=pltpu.VMEM),
        scratch_shapes=[
            pltpu.VMEM((B, Sq, HD), F32),
            pltpu.VMEM((B, Sq, HD), F32),
            pltpu.VMEM((B, Hq, Sq, 1), F32),
            pltpu.VMEM((2, B, Skv, HD), BF16),
            pltpu.VMEM((2, B, Skv, HD), BF16),
            pltpu.VMEM((2, B, Skv, HD), BF16),
            pltpu.VMEM((2, B, Skv, HD), BF16),
            pltpu.SemaphoreType.DMA((2,)),
            pltpu.SemaphoreType.DMA((2,)),
            pltpu.SemaphoreType.DMA((2,)),
            pltpu.SemaphoreType.DMA((2,)),
            pltpu.SemaphoreType.DMA((2,)),
            pltpu.SemaphoreType.DMA((2,)),
            pltpu.SemaphoreType.DMA((2,)),
            pltpu.SemaphoreType.DMA((2,)),
        ],
        compiler_params=pltpu.CompilerParams(collective_id=0),
    )(x, Wq, K2, V2, Wo)


# device time: 213968 ns/iter; 2.1195x vs baseline; 1.2829x over previous
import jax
import jax.numpy as jnp
from jax import lax
from jax.experimental import pallas as pl
from jax.experimental.pallas import tpu as pltpu

N_DEV = 32
NR = 16
NL = 15
F32 = jnp.float32
BF16 = jnp.bfloat16


def kernel(x, Wq, K_ext, V_ext, Wo):
    B, Sq, Dm = x.shape
    _, Skv, Hq, Dh = K_ext.shape
    HD = Hq * Dh
    K2 = K_ext.reshape(B, Skv, HD).astype(BF16)
    V2 = V_ext.reshape(B, Skv, HD).astype(BF16)

    def body(x_ref, wq_ref, k_ref, v_ref, wo_ref, out_ref,
             q_sc, acc_sc, l_sc,
             kR, vR, kL, vL,
             kRs, kRr, vRs, vRr, kLs, kLr, vLs, vLr):
        my = lax.axis_index("i")
        left = lax.rem(my - 1 + N_DEV, N_DEV)
        right = lax.rem(my + 1, N_DEV)

        barrier = pltpu.get_barrier_semaphore()
        pl.semaphore_signal(barrier, inc=1, device_id=(left,),
                            device_id_type=pl.DeviceIdType.MESH)
        pl.semaphore_signal(barrier, inc=1, device_id=(right,),
                            device_id_type=pl.DeviceIdType.MESH)
        pl.semaphore_wait(barrier, 2)

        for b in range(B):
            q_sc[b] = (jnp.dot(x_ref[b], wq_ref[...],
                               preferred_element_type=F32) * 0.125)
            acc_sc[b] = jnp.zeros((Sq, HD), F32)
        l_sc[...] = jnp.zeros((B, Hq, Sq, 1), F32)

        qb = lax.broadcasted_iota(jnp.int32, (Sq, Skv), 0) // 64
        kb = lax.broadcasted_iota(jnp.int32, (Sq, Skv), 1) // 64
        mask = qb == kb

        def attend(kvs):
            for b, (k32, v32) in enumerate(kvs):
                for h in range(Hq):
                    qh = q_sc[b, :, h * Dh:(h + 1) * Dh]
                    kh = k32[:, h * Dh:(h + 1) * Dh]
                    s = lax.dot_general(qh, kh, (((1,), (1,)), ((), ())),
                                        preferred_element_type=F32)
                    w = jnp.where(mask, jnp.exp(s), 0.0)
                    l_sc[b, h] += jnp.sum(w, axis=1, keepdims=True)
                    acc_sc[b, :, h * Dh:(h + 1) * Dh] += jnp.dot(
                        w, v32[:, h * Dh:(h + 1) * Dh],
                        preferred_element_type=F32)

        def attend_slot(kbuf, vbuf, s):
            attend([(kbuf[s, b].astype(F32), vbuf[s, b].astype(F32))
                    for b in range(B)])

        kR[NR] = k_ref[...]
        vR[NR] = v_ref[...]
        kL[NL] = k_ref[...]
        vL[NL] = v_ref[...]

        def mk(buf, ssem, rsem, ss, t, dev):
            return pltpu.make_async_remote_copy(
                src_ref=buf.at[ss], dst_ref=buf.at[t],
                send_sem=ssem.at[t], recv_sem=rsem.at[t],
                device_id=(dev,), device_id_type=pl.DeviceIdType.MESH)

        def start_r(ss, t):
            a = mk(kR, kRs, kRr, ss, t, right)
            b_ = mk(vR, vRs, vRr, ss, t, right)
            a.start()
            b_.start()
            return [a, b_]

        def start_l(ss, t):
            a = mk(kL, kLs, kLr, ss, t, left)
            b_ = mk(vL, vLs, vLr, ss, t, left)
            a.start()
            b_.start()
            return [a, b_]

        rd = start_r(NR, 0) + start_l(NL, 0)
        attend([(k_ref[b].astype(F32), v_ref[b].astype(F32))
                for b in range(B)])
        for d in rd:
            d.wait()

        def hop(t, carry):
            rd = start_r(t - 1, t) + start_l(t - 1, t)
            attend_slot(kR, vR, t - 1)
            attend_slot(kL, vL, t - 1)
            for d in rd:
                d.wait()
            return carry
        lax.fori_loop(1, 15, hop, 0)

        rd = start_r(14, 15)
        attend_slot(kR, vR, 14)
        attend_slot(kL, vL, 14)
        for d in rd:
            d.wait()
        attend_slot(kR, vR, 15)

        for b in range(B):
            for h in range(Hq):
                acc_sc[b, :, h * Dh:(h + 1) * Dh] = (
                    acc_sc[b, :, h * Dh:(h + 1) * Dh] / l_sc[b, h])
            out_ref[b] = jnp.dot(acc_sc[b], wo_ref[...],
                                 preferred_element_type=F32)

    return pl.pallas_call(
        body,
        out_shape=jax.ShapeDtypeStruct((B, Sq, Dm), F32),
        in_specs=[pl.BlockSpec(memory_space=pltpu.VMEM)] * 5,
        out_specs=pl.BlockSpec(memory_space=pltpu.VMEM),
        scratch_shapes=[
            pltpu.VMEM((B, Sq, HD), F32),
            pltpu.VMEM((B, Sq, HD), F32),
            pltpu.VMEM((B, Hq, Sq, 1), F32),
            pltpu.VMEM((NR + 1, B, Skv, HD), BF16),
            pltpu.VMEM((NR + 1, B, Skv, HD), BF16),
            pltpu.VMEM((NL + 1, B, Skv, HD), BF16),
            pltpu.VMEM((NL + 1, B, Skv, HD), BF16),
            pltpu.SemaphoreType.DMA((NR,)),
            pltpu.SemaphoreType.DMA((NR,)),
            pltpu.SemaphoreType.DMA((NR,)),
            pltpu.SemaphoreType.DMA((NR,)),
            pltpu.SemaphoreType.DMA((NL,)),
            pltpu.SemaphoreType.DMA((NL,)),
            pltpu.SemaphoreType.DMA((NL,)),
            pltpu.SemaphoreType.DMA((NL,)),
        ],
        compiler_params=pltpu.CompilerParams(collective_id=0),
    )(x, Wq, K2, V2, Wo)


# device time: 213488 ns/iter; 2.1243x vs baseline; 1.0022x over previous
import jax
import jax.numpy as jnp
from jax import lax
from jax.experimental import pallas as pl
from jax.experimental.pallas import tpu as pltpu

N_DEV = 32
NR = 16
NL = 15
F32 = jnp.float32
BF16 = jnp.bfloat16


def kernel(x, Wq, K_ext, V_ext, Wo):
    B, Sq, Dm = x.shape
    _, Skv, Hq, Dh = K_ext.shape
    HD = Hq * Dh
    K2 = K_ext.reshape(B, Skv, HD).astype(BF16)
    V2 = V_ext.reshape(B, Skv, HD).astype(BF16)

    def body(x_ref, wq_ref, k_ref, v_ref, wo_ref, out_ref,
             q_sc, acc_sc, l_sc,
             kR, vR, kL, vL,
             kRs, kRr, vRs, vRr, kLs, kLr, vLs, vLr):
        my = lax.axis_index("i")
        left = lax.rem(my - 1 + N_DEV, N_DEV)
        right = lax.rem(my + 1, N_DEV)

        barrier = pltpu.get_barrier_semaphore()
        pl.semaphore_signal(barrier, inc=1, device_id=(left,),
                            device_id_type=pl.DeviceIdType.MESH)
        pl.semaphore_signal(barrier, inc=1, device_id=(right,),
                            device_id_type=pl.DeviceIdType.MESH)
        pl.semaphore_wait(barrier, 2)

        for b in range(B):
            q_sc[b] = (jnp.dot(x_ref[b], wq_ref[...],
                               preferred_element_type=F32) * 0.125
                       ).astype(BF16)
            acc_sc[b] = jnp.zeros((Sq, HD), F32)
        l_sc[...] = jnp.zeros((B, Hq, Sq, 1), F32)

        qb = lax.broadcasted_iota(jnp.int32, (Sq, Skv), 0) // 64
        kb = lax.broadcasted_iota(jnp.int32, (Sq, Skv), 1) // 64
        mask = qb == kb

        def attend(kvs):
            for b, (k16, v32) in enumerate(kvs):
                for h in range(Hq):
                    qh = q_sc[b, :, h * Dh:(h + 1) * Dh]
                    kh = k16[:, h * Dh:(h + 1) * Dh]
                    s = lax.dot_general(qh, kh, (((1,), (1,)), ((), ())),
                                        preferred_element_type=F32)
                    w = jnp.where(mask, jnp.exp(s), 0.0)
                    l_sc[b, h] += jnp.sum(w, axis=1, keepdims=True)
                    acc_sc[b, :, h * Dh:(h + 1) * Dh] += jnp.dot(
                        w, v32[:, h * Dh:(h + 1) * Dh],
                        preferred_element_type=F32)

        def attend_slot(kbuf, vbuf, s):
            attend([(kbuf[s, b], vbuf[s, b].astype(F32))
                    for b in range(B)])

        kR[NR] = k_ref[...]
        vR[NR] = v_ref[...]
        kL[NL] = k_ref[...]
        vL[NL] = v_ref[...]

        def mk(buf, ssem, rsem, ss, t, dev):
            return pltpu.make_async_remote_copy(
                src_ref=buf.at[ss], dst_ref=buf.at[t],
                send_sem=ssem.at[t], recv_sem=rsem.at[t],
                device_id=(dev,), device_id_type=pl.DeviceIdType.MESH)

        def start_r(ss, t):
            a = mk(kR, kRs, kRr, ss, t, right)
            b_ = mk(vR, vRs, vRr, ss, t, right)
            a.start()
            b_.start()
            return [a, b_]

        def start_l(ss, t):
            a = mk(kL, kLs, kLr, ss, t, left)
            b_ = mk(vL, vLs, vLr, ss, t, left)
            a.start()
            b_.start()
            return [a, b_]

        rd = start_r(NR, 0) + start_l(NL, 0)
        attend([(k_ref[b], v_ref[b].astype(F32))
                for b in range(B)])
        for d in rd:
            d.wait()

        def hop(t, carry):
            rd = start_r(t - 1, t) + start_l(t - 1, t)
            attend_slot(kR, vR, t - 1)
            attend_slot(kL, vL, t - 1)
            for d in rd:
                d.wait()
            return carry
        lax.fori_loop(1, 15, hop, 0)

        rd = start_r(14, 15)
        attend_slot(kR, vR, 14)
        attend_slot(kL, vL, 14)
        for d in rd:
            d.wait()
        attend_slot(kR, vR, 15)

        for b in range(B):
            for h in range(Hq):
                acc_sc[b, :, h * Dh:(h + 1) * Dh] = (
                    acc_sc[b, :, h * Dh:(h + 1) * Dh] / l_sc[b, h])
            out_ref[b] = jnp.dot(acc_sc[b], wo_ref[...],
                                 preferred_element_type=F32)

    return pl.pallas_call(
        body,
        out_shape=jax.ShapeDtypeStruct((B, Sq, Dm), F32),
        in_specs=[pl.BlockSpec(memory_space=pltpu.VMEM)] * 5,
        out_specs=pl.BlockSpec(memory_space=pltpu.VMEM),
        scratch_shapes=[
            pltpu.VMEM((B, Sq, HD), BF16),
            pltpu.VMEM((B, Sq, HD), F32),
            pltpu.VMEM((B, Hq, Sq, 1), F32),
            pltpu.VMEM((NR + 1, B, Skv, HD), BF16),
            pltpu.VMEM((NR + 1, B, Skv, HD), BF16),
            pltpu.VMEM((NL + 1, B, Skv, HD), BF16),
            pltpu.VMEM((NL + 1, B, Skv, HD), BF16),
            pltpu.SemaphoreType.DMA((NR,)),
            pltpu.SemaphoreType.DMA((NR,)),
            pltpu.SemaphoreType.DMA((NR,)),
            pltpu.SemaphoreType.DMA((NR,)),
            pltpu.SemaphoreType.DMA((NL,)),
            pltpu.SemaphoreType.DMA((NL,)),
            pltpu.SemaphoreType.DMA((NL,)),
            pltpu.SemaphoreType.DMA((NL,)),
        ],
        compiler_params=pltpu.CompilerParams(collective_id=0),
    )(x, Wq, K2, V2, Wo)


# device time: 167915 ns/iter; 2.7008x vs baseline; 1.2714x over previous
import jax
import jax.numpy as jnp
from jax import lax
from jax.experimental import pallas as pl
from jax.experimental.pallas import tpu as pltpu

N_DEV = 32
NR = 16
NL = 15
F32 = jnp.float32
BF16 = jnp.bfloat16
F8 = jnp.float8_e4m3fn


def kernel(x, Wq, K_ext, V_ext, Wo):
    B, Sq, Dm = x.shape
    _, Skv, Hq, Dh = K_ext.shape
    HD = Hq * Dh
    K2 = K_ext.reshape(B, Skv, HD).astype(F8)
    V2 = V_ext.reshape(B, Skv, HD).astype(BF16)

    def body(x_ref, wq_ref, k_ref, v_ref, wo_ref, out_ref,
             q_sc, acc_sc, l_sc,
             kR, vR, kL, vL,
             kRs, kRr, vRs, vRr, kLs, kLr, vLs, vLr):
        my = lax.axis_index("i")
        left = lax.rem(my - 1 + N_DEV, N_DEV)
        right = lax.rem(my + 1, N_DEV)

        barrier = pltpu.get_barrier_semaphore()
        pl.semaphore_signal(barrier, inc=1, device_id=(left,),
                            device_id_type=pl.DeviceIdType.MESH)
        pl.semaphore_signal(barrier, inc=1, device_id=(right,),
                            device_id_type=pl.DeviceIdType.MESH)
        pl.semaphore_wait(barrier, 2)

        for b in range(B):
            q_sc[b] = (jnp.dot(x_ref[b], wq_ref[...],
                               preferred_element_type=F32) * 0.125
                       ).astype(BF16)
            acc_sc[b] = jnp.zeros((Sq, HD), F32)
        l_sc[...] = jnp.zeros((B, Hq, Sq, 1), F32)

        qb = lax.broadcasted_iota(jnp.int32, (Sq, Skv), 0) // 64
        kb = lax.broadcasted_iota(jnp.int32, (Sq, Skv), 1) // 64
        mask = qb == kb

        def attend(kvs):
            for b, (k16, v32) in enumerate(kvs):
                for h in range(Hq):
                    qh = q_sc[b, :, h * Dh:(h + 1) * Dh]
                    kh = k16[:, h * Dh:(h + 1) * Dh]
                    s = lax.dot_general(qh, kh, (((1,), (1,)), ((), ())),
                                        preferred_element_type=F32)
                    w = jnp.where(mask, jnp.exp(s), 0.0)
                    l_sc[b, h] += jnp.sum(w, axis=1, keepdims=True)
                    acc_sc[b, :, h * Dh:(h + 1) * Dh] += jnp.dot(
                        w, v32[:, h * Dh:(h + 1) * Dh],
                        preferred_element_type=F32)

        def attend_slot(kbuf, vbuf, s):
            attend([(kbuf[s, b].astype(BF16), vbuf[s, b].astype(F32))
                    for b in range(B)])

        kR[NR] = k_ref[...]
        vR[NR] = v_ref[...]
        kL[NL] = k_ref[...]
        vL[NL] = v_ref[...]

        def mk(buf, ssem, rsem, ss, t, dev):
            return pltpu.make_async_remote_copy(
                src_ref=buf.at[ss], dst_ref=buf.at[t],
                send_sem=ssem.at[t], recv_sem=rsem.at[t],
                device_id=(dev,), device_id_type=pl.DeviceIdType.MESH)

        def start_r(ss, t):
            a = mk(kR, kRs, kRr, ss, t, right)
            b_ = mk(vR, vRs, vRr, ss, t, right)
            a.start()
            b_.start()
            return [a, b_]

        def start_l(ss, t):
            a = mk(kL, kLs, kLr, ss, t, left)
            b_ = mk(vL, vLs, vLr, ss, t, left)
            a.start()
            b_.start()
            return [a, b_]

        rd = start_r(NR, 0) + start_l(NL, 0)
        attend([(k_ref[b].astype(BF16), v_ref[b].astype(F32))
                for b in range(B)])
        for d in rd:
            d.wait()

        def hop(t, carry):
            rd = start_r(t - 1, t) + start_l(t - 1, t)
            attend_slot(kR, vR, t - 1)
            attend_slot(kL, vL, t - 1)
            for d in rd:
                d.wait()
            return carry
        lax.fori_loop(1, 15, hop, 0)

        rd = start_r(14, 15)
        attend_slot(kR, vR, 14)
        attend_slot(kL, vL, 14)
        for d in rd:
            d.wait()
        attend_slot(kR, vR, 15)

        for b in range(B):
            for h in range(Hq):
                acc_sc[b, :, h * Dh:(h + 1) * Dh] = (
                    acc_sc[b, :, h * Dh:(h + 1) * Dh] / l_sc[b, h])
            out_ref[b] = jnp.dot(acc_sc[b], wo_ref[...],
                                 preferred_element_type=F32)

    return pl.pallas_call(
        body,
        out_shape=jax.ShapeDtypeStruct((B, Sq, Dm), F32),
        in_specs=[pl.BlockSpec(memory_space=pltpu.VMEM)] * 5,
        out_specs=pl.BlockSpec(memory_space=pltpu.VMEM),
        scratch_shapes=[
            pltpu.VMEM((B, Sq, HD), BF16),
            pltpu.VMEM((B, Sq, HD), F32),
            pltpu.VMEM((B, Hq, Sq, 1), F32),
            pltpu.VMEM((NR + 1, B, Skv, HD), F8),
            pltpu.VMEM((NR + 1, B, Skv, HD), BF16),
            pltpu.VMEM((NL + 1, B, Skv, HD), F8),
            pltpu.VMEM((NL + 1, B, Skv, HD), BF16),
            pltpu.SemaphoreType.DMA((NR,)),
            pltpu.SemaphoreType.DMA((NR,)),
            pltpu.SemaphoreType.DMA((NR,)),
            pltpu.SemaphoreType.DMA((NR,)),
            pltpu.SemaphoreType.DMA((NL,)),
            pltpu.SemaphoreType.DMA((NL,)),
            pltpu.SemaphoreType.DMA((NL,)),
            pltpu.SemaphoreType.DMA((NL,)),
        ],
        compiler_params=pltpu.CompilerParams(collective_id=0),
    )(x, Wq, K2, V2, Wo)


# device time: 128449 ns/iter; 3.5306x vs baseline; 1.3073x over previous
import jax
import jax.numpy as jnp
from jax import lax
from jax.experimental import pallas as pl
from jax.experimental.pallas import tpu as pltpu

N_DEV = 32
NR = 16
NL = 15
F32 = jnp.float32
BF16 = jnp.bfloat16
I8 = jnp.int8


def kernel(x, Wq, K_ext, V_ext, Wo):
    B, Sq, Dm = x.shape
    _, Skv, Hq, Dh = K_ext.shape
    HD = Hq * Dh
    K2 = K_ext.reshape(B, Skv, HD)
    V2 = V_ext.reshape(B, Skv, HD)

    def quant(t):
        s = jnp.max(jnp.abs(t), axis=2) / 127.0 + 1e-12
        q = jnp.clip(jnp.round(t / s[:, :, None]), -127, 127).astype(I8)
        return q, s

    K8, ks = quant(K2)
    V8, vs = quant(V2)
    SC = jnp.stack([ks, vs], axis=1)

    def body(x_ref, wq_ref, k_ref, v_ref, sc_ref, wo_ref, out_ref,
             q_sc, acc_sc, l_sc,
             kR, vR, sR, kL, vL, sL,
             kRs, kRr, vRs, vRr, sRs, sRr,
             kLs, kLr, vLs, vLr, sLs, sLr):
        my = lax.axis_index("i")
        left = lax.rem(my - 1 + N_DEV, N_DEV)
        right = lax.rem(my + 1, N_DEV)

        barrier = pltpu.get_barrier_semaphore()
        pl.semaphore_signal(barrier, inc=1, device_id=(left,),
                            device_id_type=pl.DeviceIdType.MESH)
        pl.semaphore_signal(barrier, inc=1, device_id=(right,),
                            device_id_type=pl.DeviceIdType.MESH)
        pl.semaphore_wait(barrier, 2)

        for b in range(B):
            q_sc[b] = (jnp.dot(x_ref[b], wq_ref[...],
                               preferred_element_type=F32) * 0.125
                       ).astype(BF16)
            acc_sc[b] = jnp.zeros((Sq, HD), F32)
        l_sc[...] = jnp.zeros((B, Hq, Sq, 1), F32)

        qb = lax.broadcasted_iota(jnp.int32, (Sq, Skv), 0) // 64
        kb = lax.broadcasted_iota(jnp.int32, (Sq, Skv), 1) // 64
        mask = qb == kb

        def attend(kvs):
            for b, (k16, v32, ksrow, vsrow) in enumerate(kvs):
                for h in range(Hq):
                    qh = q_sc[b, :, h * Dh:(h + 1) * Dh]
                    kh = k16[:, h * Dh:(h + 1) * Dh]
                    s = lax.dot_general(qh, kh, (((1,), (1,)), ((), ())),
                                        preferred_element_type=F32)
                    s = s * ksrow
                    w = jnp.where(mask, jnp.exp(s), 0.0)
                    l_sc[b, h] += jnp.sum(w, axis=1, keepdims=True)
                    acc_sc[b, :, h * Dh:(h + 1) * Dh] += jnp.dot(
                        w * vsrow, v32[:, h * Dh:(h + 1) * Dh],
                        preferred_element_type=F32)

        def attend_slot(kbuf, vbuf, sbuf, s):
            attend([(kbuf[s, b].astype(BF16), vbuf[s, b].astype(F32),
                     sbuf[s, b, 0:1, :], sbuf[s, b, 1:2, :])
                    for b in range(B)])

        kR[NR] = k_ref[...]
        vR[NR] = v_ref[...]
        sR[NR] = sc_ref[...]
        kL[NL] = k_ref[...]
        vL[NL] = v_ref[...]
        sL[NL] = sc_ref[...]

        def mk(buf, ssem, rsem, ss, t, dev):
            return pltpu.make_async_remote_copy(
                src_ref=buf.at[ss], dst_ref=buf.at[t],
                send_sem=ssem.at[t], recv_sem=rsem.at[t],
                device_id=(dev,), device_id_type=pl.DeviceIdType.MESH)

        def start_r(ss, t):
            ds = [mk(kR, kRs, kRr, ss, t, right),
                  mk(vR, vRs, vRr, ss, t, right),
                  mk(sR, sRs, sRr, ss, t, right)]
            for d in ds:
                d.start()
            return ds

        def start_l(ss, t):
            ds = [mk(kL, kLs, kLr, ss, t, left),
                  mk(vL, vLs, vLr, ss, t, left),
                  mk(sL, sLs, sLr, ss, t, left)]
            for d in ds:
                d.start()
            return ds

        rd = start_r(NR, 0) + start_l(NL, 0)
        attend([(k_ref[b].astype(BF16), v_ref[b].astype(F32),
                 sc_ref[b, 0:1, :], sc_ref[b, 1:2, :])
                for b in range(B)])
        for d in rd:
            d.wait()

        def hop(t, carry):
            rd = start_r(t - 1, t) + start_l(t - 1, t)
            attend_slot(kR, vR, sR, t - 1)
            attend_slot(kL, vL, sL, t - 1)
            for d in rd:
                d.wait()
            return carry
        lax.fori_loop(1, 15, hop, 0)

        rd = start_r(14, 15)
        attend_slot(kR, vR, sR, 14)
        attend_slot(kL, vL, sL, 14)
        for d in rd:
            d.wait()
        attend_slot(kR, vR, sR, 15)

        for b in range(B):
            for h in range(Hq):
                acc_sc[b, :, h * Dh:(h + 1) * Dh] = (
                    acc_sc[b, :, h * Dh:(h + 1) * Dh] / l_sc[b, h])
            out_ref[b] = jnp.dot(acc_sc[b], wo_ref[...],
                                 preferred_element_type=F32)

    return pl.pallas_call(
        body,
        out_shape=jax.ShapeDtypeStruct((B, Sq, Dm), F32),
        in_specs=[pl.BlockSpec(memory_space=pltpu.VMEM)] * 6,
        out_specs=pl.BlockSpec(memory_space=pltpu.VMEM),
        scratch_shapes=[
            pltpu.VMEM((B, Sq, HD), BF16),
            pltpu.VMEM((B, Sq, HD), F32),
            pltpu.VMEM((B, Hq, Sq, 1), F32),
            pltpu.VMEM((NR + 1, B, Skv, HD), I8),
            pltpu.VMEM((NR + 1, B, Skv, HD), I8),
            pltpu.VMEM((NR + 1, B, 2, Skv), F32),
            pltpu.VMEM((NL + 1, B, Skv, HD), I8),
            pltpu.VMEM((NL + 1, B, Skv, HD), I8),
            pltpu.VMEM((NL + 1, B, 2, Skv), F32),
            pltpu.SemaphoreType.DMA((NR,)),
            pltpu.SemaphoreType.DMA((NR,)),
            pltpu.SemaphoreType.DMA((NR,)),
            pltpu.SemaphoreType.DMA((NR,)),
            pltpu.SemaphoreType.DMA((NR,)),
            pltpu.SemaphoreType.DMA((NR,)),
            pltpu.SemaphoreType.DMA((NL,)),
            pltpu.SemaphoreType.DMA((NL,)),
            pltpu.SemaphoreType.DMA((NL,)),
            pltpu.SemaphoreType.DMA((NL,)),
            pltpu.SemaphoreType.DMA((NL,)),
            pltpu.SemaphoreType.DMA((NL,)),
        ],
        compiler_params=pltpu.CompilerParams(collective_id=0),
    )(x, Wq, K8, V8, SC, Wo)
